# Initial kernel scaffold; baseline (speedup 1.0000x reference)
import jax
import jax.numpy as jnp
from jax.experimental import pallas as pl


def kernel(seqlen, x):
    raise NotImplementedError("write your pallas kernel here")


# speedup vs baseline: 1.2333x; 1.2333x over previous
import functools

import jax
import jax.numpy as jnp
from jax import lax
from jax.experimental import pallas as pl
from jax.experimental.pallas import tpu as pltpu
from jax.experimental.pallas import tpu_sc as plsc

SEQ_MAX_LEN = 200
N_INPUT = 128
BATCH = 4096

_INFO = plsc.get_sparse_core_info()
_NC = _INFO.num_cores
_NS = _INFO.num_subcores
_NW = _NC * _NS
_L = _INFO.num_lanes
_B_PER_W = BATCH // _NW


@functools.partial(
    pl.kernel,
    out_type=jax.ShapeDtypeStruct((BATCH, N_INPUT), jnp.float32),
    mesh=plsc.VectorSubcoreMesh(core_axis_name="c", subcore_axis_name="s"),
    scratch_types=[
        pltpu.VMEM((_B_PER_W,), jnp.int32),
        pltpu.VMEM((_B_PER_W,), jnp.int32),
        pltpu.VMEM((_B_PER_W, N_INPUT), jnp.float32),
        pltpu.SemaphoreType.DMA,
    ],
)
def _gather_last(table_hbm, seq_hbm, out_hbm, seq_v, idx_v, rows_v, sem):
    wid = lax.axis_index("s") * _NC + lax.axis_index("c")
    base = wid * _B_PER_W
    pltpu.sync_copy(seq_hbm.at[pl.ds(base, _B_PER_W)], seq_v)
    for j in range(_B_PER_W // _L):
        s = seq_v[pl.ds(j * _L, _L)]
        pos = lax.broadcasted_iota(jnp.int32, (_L,), 0) + (base + j * _L)
        idx_v[pl.ds(j * _L, _L)] = pos * SEQ_MAX_LEN + (s - 1)
    pltpu.async_copy(table_hbm.at[idx_v], rows_v, sem).wait()
    pltpu.sync_copy(rows_v, out_hbm.at[pl.ds(base, _B_PER_W)])


def kernel(seqlen, x):
    table = x.reshape(BATCH * SEQ_MAX_LEN, N_INPUT)
    seq32 = seqlen.reshape(-1).astype(jnp.int32)
    return _gather_last(table, seq32)
